# Initial kernel scaffold; baseline (speedup 1.0000x reference)
import jax
import jax.numpy as jnp
from jax.experimental import pallas as pl


def kernel(x, edge_index, num_nodes, W1, b1, W2, b2, prop_weights):
    raise NotImplementedError("write your pallas kernel here")


# speedup vs baseline: 18.0391x; 18.0391x over previous
import functools

import jax
import jax.numpy as jnp
from jax import lax
from jax.experimental import pallas as pl
from jax.experimental.pallas import tpu as pltpu
from jax.experimental.pallas import tpu_sc as plsc

N = 10000
FEAT = 128
K = 10
NC = 2
NS = 16
L = 16
CPS = FEAT // NC
NPAD = 10240
SR = NPAD // NS
RCK = 80
ZCK = 128
PAD_ROW = N
EC = 128
EBLK = 16


def _rsqrt16(d):
    i = plsc.bitcast(d, jnp.int32)
    i = jnp.int32(0x5F3759DF) - (i >> 1)
    y = plsc.bitcast(i, jnp.float32)
    for _ in range(3):
        y = y * (1.5 - 0.5 * d * y * y)
    return y


def _bcast(ref, idx):
    return plsc.load_gather(ref, [jnp.full((L,), idx, jnp.int32)])


def _mlp_body(x_ref, w1_ref, b1_ref, w2_ref, b2_ref, o_ref):
    h = lax.dot_general(x_ref[...], w1_ref[...], (((1,), (1,)), ((), ())),
                        preferred_element_type=jnp.float32)
    h = jnp.maximum(h + b1_ref[...], 0.0)
    o_ref[...] = lax.dot_general(h, w2_ref[...], (((1,), (1,)), ((), ())),
                                 preferred_element_type=jnp.float32) + b2_ref[...]


def _mlp(x, W1, b1, W2, b2):
    n, in_c = x.shape
    hid = W1.shape[0]
    out_c = W2.shape[0]
    blk = 1000
    return pl.pallas_call(
        _mlp_body,
        grid=(n // blk,),
        in_specs=[
            pl.BlockSpec((blk, in_c), lambda i: (i, 0)),
            pl.BlockSpec((hid, in_c), lambda i: (0, 0)),
            pl.BlockSpec((1, hid), lambda i: (0, 0)),
            pl.BlockSpec((out_c, hid), lambda i: (0, 0)),
            pl.BlockSpec((1, out_c), lambda i: (0, 0)),
        ],
        out_specs=pl.BlockSpec((blk, out_c), lambda i: (i, 0)),
        out_shape=jax.ShapeDtypeStruct((n, out_c), jnp.float32),
    )(x, W1, b1.reshape(1, -1), W2, b2.reshape(1, -1))


def _make_prop(ch):
    eblks = ch // EBLK
    mesh = plsc.VectorSubcoreMesh(core_axis_name="c", subcore_axis_name="s")

    @functools.partial(
        pl.kernel,
        out_type=jax.ShapeDtypeStruct((N, FEAT), jnp.float32),
        mesh=mesh,
        scratch_types=[
            pltpu.VMEM_SHARED((NPAD, CPS), jnp.float32),
            pltpu.VMEM_SHARED((NPAD, CPS), jnp.float32),
            pltpu.VMEM((EBLK, EC), jnp.int32),
            pltpu.VMEM((EBLK, EC), jnp.int32),
            pltpu.VMEM((EC, CPS), jnp.float32),
            pltpu.VMEM((EC, CPS), jnp.float32),
            pltpu.VMEM((EC, CPS), jnp.float32),
            pltpu.VMEM((EC, CPS), jnp.float32),
            pltpu.VMEM((RCK, CPS), jnp.float32),
            pltpu.VMEM((RCK, CPS), jnp.float32),
            pltpu.VMEM((SR,), jnp.float32),
            pltpu.VMEM((L,), jnp.float32),
            pltpu.SemaphoreType.DMA,
            pltpu.SemaphoreType.DMA,
            pltpu.SemaphoreType.DMA,
            pltpu.SemaphoreType.DMA,
            pltpu.SemaphoreType.DMA,
            pltpu.SemaphoreType.DMA,
            pltpu.SemaphoreType.DMA,
            pltpu.SemaphoreType.DMA,
        ],
        compiler_params=pltpu.CompilerParams(
            use_tc_tiling_on_sc=False, needs_layout_passes=False),
    )
    def prop(h_hbm, rows_hbm, cols_hbm, pw_hbm, out_hbm,
             z_sh, acc_sh, ebuf_r, ebuf_c, gb0, gb1, gb2, gb3,
             nbuf_a, nbuf_o, dinv_v, pw_v,
             sg0, sg1, sg2, sg3, ss0, ss1, ss2, ss3):
        c = lax.axis_index("c")
        s = lax.axis_index("s")
        col0 = c * CPS
        r0 = s * SR
        nrc = jnp.where(s == NS - 1, (N - (NS - 1) * SR) // RCK, SR // RCK)
        gbufs = (gb0, gb1, gb2, gb3)
        sgs = (sg0, sg1, sg2, sg3)
        sss = (ss0, ss1, ss2, ss3)

        pltpu.sync_copy(pw_hbm, pw_v)

        def fill_gb0(v):
            def body(r, carry):
                for j in range(CPS // L):
                    gb0[r, pl.ds(j * L, L)] = jnp.full((L,), v, jnp.float32)
                return carry
            lax.fori_loop(0, ZCK, body, 0)

        fill_gb0(0.0)
        for b in range(SR // ZCK):
            sl = pl.ds(r0 + b * ZCK, ZCK)
            pltpu.sync_copy(gb0, z_sh.at[sl, :])
            pltpu.sync_copy(gb0, acc_sh.at[sl, :])
        plsc.subcore_barrier()

        fill_gb0(1.0)

        def deg_blk(bk, carry):
            pltpu.sync_copy(cols_hbm.at[s, pl.ds(bk * EBLK, EBLK)], ebuf_c)
            descs = [pltpu.async_copy(gb0, acc_sh.at[ebuf_c.at[j]],
                                      sss[j % 4], add=True)
                     for j in range(EBLK)]
            for d in descs:
                d.wait()
            return carry
        lax.fori_loop(0, eblks, deg_blk, 0)
        plsc.subcore_barrier()

        for b in range(SR // ZCK):
            base = r0 + b * ZCK
            pltpu.sync_copy(acc_sh.at[pl.ds(base, ZCK), :], gb1)
            for t in range(ZCK // L):
                ridx = lax.iota(jnp.int32, L) + (t * L)
                deg = plsc.load_gather(gb1, [ridx, jnp.full((L,), 1, jnp.int32)])
                deg = deg + 1.0
                y = _rsqrt16(deg)
                y = jnp.where(ridx + base < N, y, 0.0)
                dinv_v[pl.ds(b * ZCK + t * L, L)] = y

        fill_gb0(0.0)
        for b in range(SR // ZCK):
            pltpu.sync_copy(gb0, acc_sh.at[pl.ds(r0 + b * ZCK, ZCK), :])

        pw0 = _bcast(pw_v, 1)

        def h_body(i, carry):
            base = r0 + i * RCK
            pltpu.sync_copy(h_hbm.at[pl.ds(base, RCK), pl.ds(col0, CPS)], nbuf_o)

            def h_row(r, rc):
                dbc = _bcast(dinv_v, i * RCK + r)
                for j in range(CPS // L):
                    sl = pl.ds(j * L, L)
                    hv = nbuf_o[r, sl]
                    nbuf_a[r, sl] = dbc * hv
                    nbuf_o[r, sl] = pw0 * hv
                return rc
            lax.fori_loop(0, RCK, h_row, 0)
            dsl = pl.ds(base, RCK)
            pltpu.sync_copy(nbuf_a, z_sh.at[dsl, :])
            pltpu.sync_copy(nbuf_a, acc_sh.at[dsl, :])
            pltpu.sync_copy(nbuf_o, out_hbm.at[dsl, pl.ds(col0, CPS)])
            return carry
        lax.fori_loop(0, nrc, h_body, 0)
        plsc.subcore_barrier()

        for k in range(1, K + 1):
            def edge_blk(bk, carry):
                pltpu.sync_copy(rows_hbm.at[s, pl.ds(bk * EBLK, EBLK)], ebuf_r)
                pltpu.sync_copy(cols_hbm.at[s, pl.ds(bk * EBLK, EBLK)], ebuf_c)
                gd = {
                    0: pltpu.async_copy(z_sh.at[ebuf_c.at[0]], gb0, sg0),
                    1: pltpu.async_copy(z_sh.at[ebuf_c.at[1]], gb1, sg1),
                }
                sd = {}
                for j in range(EBLK):
                    b = j % 4
                    gd[j].wait()
                    sd[j] = pltpu.async_copy(gbufs[b], acc_sh.at[ebuf_r.at[j]],
                                             sss[b], add=True)
                    m = j + 2
                    if m < EBLK:
                        if j >= 2:
                            sd[j - 2].wait()
                        gd[m] = pltpu.async_copy(z_sh.at[ebuf_c.at[m]],
                                                 gbufs[m % 4], sgs[m % 4])
                for jj in range(EBLK - 4, EBLK):
                    sd[jj].wait()
                return carry
            lax.fori_loop(0, eblks, edge_blk, 0)
            plsc.subcore_barrier()

            pwk = _bcast(pw_v, k + 1)
            last = k == K

            def out_body(i, carry):
                base = r0 + i * RCK
                pltpu.sync_copy(acc_sh.at[pl.ds(base, RCK), :], nbuf_a)
                pltpu.sync_copy(out_hbm.at[pl.ds(base, RCK), pl.ds(col0, CPS)],
                                nbuf_o)

                def out_row(r, rc):
                    dbc = _bcast(dinv_v, i * RCK + r)
                    for j in range(CPS // L):
                        sl = pl.ds(j * L, L)
                        t = dbc * nbuf_a[r, sl]
                        nbuf_o[r, sl] = nbuf_o[r, sl] + pwk * t
                        if not last:
                            nbuf_a[r, sl] = dbc * t
                    return rc
                lax.fori_loop(0, RCK, out_row, 0)

                pltpu.sync_copy(nbuf_o, out_hbm.at[pl.ds(base, RCK),
                                                   pl.ds(col0, CPS)])
                if not last:
                    pltpu.sync_copy(nbuf_a, z_sh.at[pl.ds(base, RCK), :])
                    pltpu.sync_copy(nbuf_a, acc_sh.at[pl.ds(base, RCK), :])
                return carry
            lax.fori_loop(0, nrc, out_body, 0)
            if not last:
                plsc.subcore_barrier()

    return prop


def kernel(x, edge_index, num_nodes, W1, b1, W2, b2, prop_weights):
    h = _mlp(x, W1, b1, W2, b2)

    e = edge_index.shape[1]
    ept = -(-e // NS)
    ch = -(-ept // (EC * EBLK)) * EBLK
    tot = NS * ch * EC
    pad = tot - e
    row = edge_index[0].astype(jnp.int32)
    col = edge_index[1].astype(jnp.int32)
    padv = PAD_ROW + jnp.arange(pad, dtype=jnp.int32) % (NPAD - N)
    rows_p = jnp.concatenate([row, padv]).reshape(NS, ch, EC)
    cols_p = jnp.concatenate([col, padv]).reshape(NS, ch, EC)
    pw = jnp.zeros((L,), jnp.float32).at[1: 1 + prop_weights.shape[0]].set(
        prop_weights.astype(jnp.float32))

    return _make_prop(ch)(h, rows_p, cols_p, pw)
